# baseline (speedup 1.0000x reference)
import jax
import jax.numpy as jnp
from jax.experimental import pallas as pl
from jax.experimental.pallas import tpu as pltpu

B = 1024
K = 2048
N = 100000
BN = 1024
NBLK = 98
TAIL = N - (NBLK - 1) * BN
NSTEP = NBLK // 2
NC = 4
LROWS = BN // NC
OROWS = B // NC


def _body(x_hbm, lut_hbm, out_hbm, x_v, lutA, lutB, outA, outB, out_t,
          sx, sa, sb, soa, sob, st):
    i = pl.program_id(0)
    j0 = 2 * i
    j1 = 2 * i + 1
    last = NSTEP - 1

    def lut_in(j, buf, sems, c):
        return pltpu.make_async_copy(
            lut_hbm.at[pl.ds(j * BN + c * LROWS, LROWS), :],
            buf.at[pl.ds(c * LROWS, LROWS), :], sems.at[c])

    def lut_in_tail(buf, sem):
        return pltpu.make_async_copy(
            lut_hbm.at[pl.ds((NBLK - 1) * BN, TAIL), :],
            buf.at[pl.ds(0, TAIL), :], sem)

    def out_w(j, buf, sems, c):
        return pltpu.make_async_copy(
            buf.at[pl.ds(c * OROWS, OROWS), :],
            out_hbm.at[pl.ds(c * OROWS, OROWS), pl.ds(j * BN, BN)],
            sems.at[c])

    def out_w_tail(sem):
        return pltpu.make_async_copy(
            out_t, out_hbm.at[:, pl.ds((NBLK - 1) * BN, TAIL)], sem)

    @pl.when(i == 0)
    def _():
        xcopy = pltpu.make_async_copy(x_hbm, x_v, sx)
        xcopy.start()
        for c in range(NC):
            lut_in(j0, lutA, sa, c).start()
        xcopy.wait()

    @pl.when(i < last)
    def _():
        for c in range(NC):
            lut_in(j1, lutB, sb, c).start()

    @pl.when(i == last)
    def _():
        lut_in_tail(lutB, st).start()

    @pl.when(i > 0)
    def _():
        for c in range(NC):
            out_w(j0 - 2, outA, soa, c).wait()

    for c in range(NC):
        lut_in(j0, lutA, sa, c).wait()

    xb = x_v[...]
    outA[...] = jax.lax.dot_general(
        xb, lutA[...].astype(jnp.bfloat16),
        dimension_numbers=(((1,), (1,)), ((), ())),
        preferred_element_type=jnp.float32)
    for c in range(NC):
        out_w(j0, outA, soa, c).start()

    @pl.when(i < last)
    def _():
        for c in range(NC):
            lut_in(j0 + 2, lutA, sa, c).start()

    @pl.when(i > 0)
    def _():
        for c in range(NC):
            out_w(j1 - 2, outB, sob, c).wait()

    @pl.when(i < last)
    def _():
        for c in range(NC):
            lut_in(j1, lutB, sb, c).wait()
        outB[...] = jax.lax.dot_general(
            xb, lutB[...].astype(jnp.bfloat16),
            dimension_numbers=(((1,), (1,)), ((), ())),
            preferred_element_type=jnp.float32)
        for c in range(NC):
            out_w(j1, outB, sob, c).start()

    @pl.when(i == last)
    def _():
        lut_in_tail(lutB, st).wait()
        out_t[...] = jax.lax.dot_general(
            xb, lutB[pl.ds(0, TAIL), :].astype(jnp.bfloat16),
            dimension_numbers=(((1,), (1,)), ((), ())),
            preferred_element_type=jnp.float32)
        out_w_tail(st).start()
        for c in range(NC):
            out_w(j0, outA, soa, c).wait()
        out_w_tail(st).wait()


def kernel(x, person_id, LUT):
    del person_id
    xb = x.astype(jnp.bfloat16)
    return pl.pallas_call(
        _body,
        grid=(NSTEP,),
        in_specs=[
            pl.BlockSpec(memory_space=pl.ANY),
            pl.BlockSpec(memory_space=pl.ANY),
        ],
        out_specs=pl.BlockSpec(memory_space=pl.ANY),
        out_shape=jax.ShapeDtypeStruct((B, N), jnp.float32),
        scratch_shapes=[
            pltpu.VMEM((B, K), jnp.bfloat16),
            pltpu.VMEM((BN, K), jnp.float32),
            pltpu.VMEM((BN, K), jnp.float32),
            pltpu.VMEM((B, BN), jnp.float32),
            pltpu.VMEM((B, BN), jnp.float32),
            pltpu.VMEM((B, TAIL), jnp.float32),
            pltpu.SemaphoreType.DMA,
            pltpu.SemaphoreType.DMA((NC,)),
            pltpu.SemaphoreType.DMA((NC,)),
            pltpu.SemaphoreType.DMA((NC,)),
            pltpu.SemaphoreType.DMA((NC,)),
            pltpu.SemaphoreType.DMA,
        ],
        compiler_params=pltpu.CompilerParams(
            dimension_semantics=("arbitrary",),
        ),
    )(xb, LUT)


# speedup vs baseline: 1.7484x; 1.7484x over previous
import jax
import jax.numpy as jnp
from jax.experimental import pallas as pl
from jax.experimental.pallas import tpu as pltpu

B = 1024
K = 2048
N = 100000
BN = 1024
NBLK = 98
TAIL = N - (NBLK - 1) * BN
NSTEP = NBLK // 2


def _body(x_hbm, lut_hbm, out_hbm, x_v, lutA, lutB, outA, outB, out_t,
          sx, sa, sb, soa, sob, st):
    last = NSTEP - 1

    def lut_in(j, buf, sem):
        return pltpu.make_async_copy(
            lut_hbm.at[pl.ds(j * BN, BN), :], buf, sem)

    def lut_in_tail(buf, sem):
        return pltpu.make_async_copy(
            lut_hbm.at[pl.ds((NBLK - 1) * BN, TAIL), :],
            buf.at[pl.ds(0, TAIL), :], sem)

    def out_w(j, buf, sem):
        return pltpu.make_async_copy(
            buf, out_hbm.at[:, pl.ds(j * BN, BN)], sem)

    def out_w_tail(sem):
        return pltpu.make_async_copy(
            out_t, out_hbm.at[:, pl.ds((NBLK - 1) * BN, TAIL)], sem)

    xcopy = pltpu.make_async_copy(x_hbm, x_v, sx)
    xcopy.start()
    lut_in(0, lutA, sa).start()
    xcopy.wait()
    xb = x_v[...]

    def step(i, carry):
        j0 = 2 * i
        j1 = 2 * i + 1

        @pl.when(i < last)
        def _():
            lut_in(j1, lutB, sb).start()

        @pl.when(i == last)
        def _():
            lut_in_tail(lutB, st).start()

        @pl.when(i > 0)
        def _():
            out_w(j0 - 2, outA, soa).wait()

        lut_in(j0, lutA, sa).wait()
        outA[...] = jax.lax.dot_general(
            xb, lutA[...].astype(jnp.bfloat16),
            dimension_numbers=(((1,), (1,)), ((), ())),
            preferred_element_type=jnp.float32)
        out_w(j0, outA, soa).start()

        @pl.when(i < last)
        def _():
            lut_in(j0 + 2, lutA, sa).start()

        @pl.when(i > 0)
        def _():
            out_w(j1 - 2, outB, sob).wait()

        @pl.when(i < last)
        def _():
            lut_in(j1, lutB, sb).wait()
            outB[...] = jax.lax.dot_general(
                xb, lutB[...].astype(jnp.bfloat16),
                dimension_numbers=(((1,), (1,)), ((), ())),
                preferred_element_type=jnp.float32)
            out_w(j1, outB, sob).start()

        @pl.when(i == last)
        def _():
            lut_in_tail(lutB, st).wait()
            out_t[...] = jax.lax.dot_general(
                xb, lutB[pl.ds(0, TAIL), :].astype(jnp.bfloat16),
                dimension_numbers=(((1,), (1,)), ((), ())),
                preferred_element_type=jnp.float32)
            out_w_tail(st).start()
            out_w(j0, outA, soa).wait()
            out_w_tail(st).wait()

        return carry

    jax.lax.fori_loop(0, NSTEP, step, 0)


def kernel(x, person_id, LUT):
    del person_id
    xb = x.astype(jnp.bfloat16)
    return pl.pallas_call(
        _body,
        in_specs=[
            pl.BlockSpec(memory_space=pl.ANY),
            pl.BlockSpec(memory_space=pl.ANY),
        ],
        out_specs=pl.BlockSpec(memory_space=pl.ANY),
        out_shape=jax.ShapeDtypeStruct((B, N), jnp.float32),
        scratch_shapes=[
            pltpu.VMEM((B, K), jnp.bfloat16),
            pltpu.VMEM((BN, K), jnp.float32),
            pltpu.VMEM((BN, K), jnp.float32),
            pltpu.VMEM((B, BN), jnp.float32),
            pltpu.VMEM((B, BN), jnp.float32),
            pltpu.VMEM((B, TAIL), jnp.float32),
            pltpu.SemaphoreType.DMA,
            pltpu.SemaphoreType.DMA,
            pltpu.SemaphoreType.DMA,
            pltpu.SemaphoreType.DMA,
            pltpu.SemaphoreType.DMA,
            pltpu.SemaphoreType.DMA,
        ],
    )(xb, LUT)


# speedup vs baseline: 1.7739x; 1.0146x over previous
import jax
import jax.numpy as jnp
from jax.experimental import pallas as pl
from jax.experimental.pallas import tpu as pltpu

B = 1024
K = 2048
N = 100000
BN = 1000
NBLK = N // BN
NSTEP = NBLK // 4


def _body(x_hbm, lut_hbm, out_hbm, x_v, l0, l1, l2, l3, o0, o1,
          sx, sl0, sl1, sl2, sl3, so0, so1):
    lbufs = (l0, l1, l2, l3)
    lsems = (sl0, sl1, sl2, sl3)

    def lut_in(j, c):
        return pltpu.make_async_copy(
            lut_hbm.at[pl.ds(j * BN, BN), :], lbufs[c], lsems[c])

    def out_w(j, buf, sem):
        return pltpu.make_async_copy(
            buf, out_hbm.at[pl.ds(j * BN, BN), :], sem)

    xcopy = pltpu.make_async_copy(x_hbm, x_v, sx)
    xcopy.start()
    for c in range(4):
        lut_in(c, c).start()
    xcopy.wait()
    xb = x_v[...]

    def dot(lbuf):
        return jax.lax.dot_general(
            lbuf[...].astype(jnp.bfloat16), xb,
            dimension_numbers=(((1,), (1,)), ((), ())),
            preferred_element_type=jnp.float32)

    def step(i, carry):
        j0 = 4 * i

        @pl.when(i > 0)
        def _():
            out_w(j0 - 2, o0, so0).wait()

        lut_in(j0, 0).wait()
        o0[...] = dot(l0)
        out_w(j0, o0, so0).start(priority=1)

        @pl.when(i + 1 < NSTEP)
        def _():
            lut_in(j0 + 4, 0).start()

        @pl.when(i > 0)
        def _():
            out_w(j0 - 1, o1, so1).wait()

        lut_in(j0 + 1, 1).wait()
        o1[...] = dot(l1)
        out_w(j0 + 1, o1, so1).start(priority=1)

        @pl.when(i + 1 < NSTEP)
        def _():
            lut_in(j0 + 5, 1).start()

        out_w(j0, o0, so0).wait()
        lut_in(j0 + 2, 2).wait()
        o0[...] = dot(l2)
        out_w(j0 + 2, o0, so0).start(priority=1)

        @pl.when(i + 1 < NSTEP)
        def _():
            lut_in(j0 + 6, 2).start()

        out_w(j0 + 1, o1, so1).wait()
        lut_in(j0 + 3, 3).wait()
        o1[...] = dot(l3)
        out_w(j0 + 3, o1, so1).start(priority=1)

        @pl.when(i + 1 < NSTEP)
        def _():
            lut_in(j0 + 7, 3).start()

        @pl.when(i + 1 == NSTEP)
        def _():
            out_w(j0 + 2, o0, so0).wait()
            out_w(j0 + 3, o1, so1).wait()

        return carry

    jax.lax.fori_loop(0, NSTEP, step, 0)


def kernel(x, person_id, LUT):
    del person_id
    xb = x.astype(jnp.bfloat16)
    out_t = pl.pallas_call(
        _body,
        in_specs=[
            pl.BlockSpec(memory_space=pl.ANY),
            pl.BlockSpec(memory_space=pl.ANY),
        ],
        out_specs=pl.BlockSpec(memory_space=pl.ANY),
        out_shape=jax.ShapeDtypeStruct((N, B), jnp.float32),
        scratch_shapes=[
            pltpu.VMEM((B, K), jnp.bfloat16),
            pltpu.VMEM((BN, K), jnp.float32),
            pltpu.VMEM((BN, K), jnp.float32),
            pltpu.VMEM((BN, K), jnp.float32),
            pltpu.VMEM((BN, K), jnp.float32),
            pltpu.VMEM((BN, B), jnp.float32),
            pltpu.VMEM((BN, B), jnp.float32),
            pltpu.SemaphoreType.DMA,
            pltpu.SemaphoreType.DMA,
            pltpu.SemaphoreType.DMA,
            pltpu.SemaphoreType.DMA,
            pltpu.SemaphoreType.DMA,
            pltpu.SemaphoreType.DMA,
            pltpu.SemaphoreType.DMA,
        ],
    )(xb, LUT)
    return out_t.T
